# baseline (speedup 1.0000x reference)
import jax
import jax.numpy as jnp
import numpy as np
from jax import lax
from jax.experimental import pallas as pl
from jax.experimental.pallas import tpu as pltpu
from jax.experimental.pallas import tpu_sc as plsc

N = 10000
NP = 10240
E = 160000
HID = 256
NT = 16
NC = 2
ROWS_T = NP // NT
EP_T = E // NT
BATCH = 80
NB = EP_T // BATCH
RBLK = 1000
NRB = N // RBLK

try:
    _PERM = np.asarray(
        jax.device_get(jax.random.permutation(jax.random.key(42), N)),
        dtype=np.int32)
    _PERM_INV = np.argsort(_PERM).astype(np.int32)
except Exception:
    _PERM = None
    _PERM_INV = None


def _perm_arrays():
    if _PERM is not None:
        return jnp.asarray(_PERM), None
    perm = jax.random.permutation(jax.random.key(42), N).astype(jnp.int32)
    return perm, None


_MESH = plsc.VectorSubcoreMesh(core_axis_name="c", subcore_axis_name="s")
_SC_PARAMS = pltpu.CompilerParams(needs_layout_passes=False)


def _prep_body(feat_hbm, permb_hbm, edges3_hbm, z128_hbm, ones_hbm,
               deg2_hbm, xp_hbm,
               pidx_v, prow_v, idx_v, ones_v, acc, psem):
    c = lax.axis_index("c")
    s = lax.axis_index("s")
    wid = c * NT + s

    pltpu.sync_copy(permb_hbm.at[c, s], pidx_v)
    for b in range(4):
        pltpu.async_copy(feat_hbm.at[pidx_v.at[b]], prow_v, psem).wait()
        pltpu.sync_copy(prow_v,
                        xp_hbm.at[pl.ds(wid * 320 + b * BATCH, BATCH)])

    pltpu.sync_copy(ones_hbm, ones_v)
    pltpu.sync_copy(z128_hbm.at[pl.ds(s * ROWS_T, ROWS_T)],
                    acc.at[pl.ds(s * ROWS_T, ROWS_T)])
    pltpu.sync_copy(edges3_hbm.at[c, s], idx_v)
    plsc.subcore_barrier()

    def dbody(b, carry):
        pltpu.sync_copy(ones_v, acc.at[idx_v.at[b]], add=True)
        return carry

    lax.fori_loop(0, NB, dbody, 0)
    plsc.subcore_barrier()
    pltpu.sync_copy(acc.at[pl.ds(s * ROWS_T, ROWS_T)],
                    deg2_hbm.at[c, pl.ds(s * ROWS_T, ROWS_T)])


_prep = pl.kernel(
    _prep_body,
    out_type=[
        jax.ShapeDtypeStruct((NC, NP, 128), jnp.float32),
        jax.ShapeDtypeStruct((NP, HID), jnp.float32),
    ],
    mesh=_MESH,
    scratch_types=[
        pltpu.VMEM((4, BATCH), jnp.int32),
        pltpu.VMEM((BATCH, HID), jnp.float32),
        pltpu.VMEM((NB, BATCH), jnp.int32),
        pltpu.VMEM((BATCH, 128), jnp.float32),
        pltpu.VMEM_SHARED((NP, 128), jnp.float32),
        pltpu.SemaphoreType.DMA,
    ],
    compiler_params=_SC_PARAMS,
)


def _edge_body(t_hbm, srcs_hbm, dst3_hbm, z128_hbm,
               agg_hbm,
               src_v, dst_v, rows0, rows1, acc, sem0, sem1):
    c = lax.axis_index("c")
    s = lax.axis_index("s")
    pltpu.sync_copy(srcs_hbm.at[pl.ds(c * E + s * EP_T, EP_T)], src_v)
    pltpu.sync_copy(dst3_hbm.at[s], dst_v)
    rows = (rows0, rows1)
    sems = (sem0, sem1)

    for k in range(2):
        tbl = t_hbm.at[k]
        pltpu.sync_copy(z128_hbm.at[pl.ds(s * ROWS_T, ROWS_T)],
                        acc.at[pl.ds(s * ROWS_T, ROWS_T)])
        plsc.subcore_barrier()

        pltpu.async_copy(tbl.at[src_v.at[pl.ds(0, BATCH)]], rows0, sem0)

        def pair(i, carry):
            for j in range(2):
                b = i * 2 + j

                @pl.when(b + 1 < NB)
                def _():
                    pltpu.async_copy(
                        tbl.at[src_v.at[pl.ds((b + 1) * BATCH, BATCH)]],
                        rows[1 - j], sems[1 - j])

                pltpu.make_async_copy(
                    tbl.at[src_v.at[pl.ds(b * BATCH, BATCH)]],
                    rows[j], sems[j]).wait()
                pltpu.sync_copy(rows[j], acc.at[dst_v.at[b]], add=True)
            return carry

        lax.fori_loop(0, NB // 2, pair, 0)
        b_last = NB - 1
        pltpu.make_async_copy(
            tbl.at[src_v.at[pl.ds(b_last * BATCH, BATCH)]],
            rows0, sem0).wait()
        pltpu.sync_copy(rows0, acc.at[dst_v.at[b_last]], add=True)

        plsc.subcore_barrier()
        row0 = c * NP + s * ROWS_T
        pltpu.sync_copy(acc.at[pl.ds(s * ROWS_T, ROWS_T)],
                        agg_hbm.at[k, pl.ds(row0, ROWS_T)])
        plsc.subcore_barrier()


_edge_pass = pl.kernel(
    _edge_body,
    out_type=jax.ShapeDtypeStruct((2, 2 * NP, 128), jnp.float32),
    mesh=_MESH,
    scratch_types=[
        pltpu.VMEM((EP_T,), jnp.int32),
        pltpu.VMEM((NB, BATCH), jnp.int32),
        pltpu.VMEM((BATCH, 128), jnp.float32),
        pltpu.VMEM((BATCH, 128), jnp.float32),
        pltpu.VMEM_SHARED((NP, 128), jnp.float32),
        pltpu.SemaphoreType.DMA,
        pltpu.SemaphoreType.DMA,
    ],
    compiler_params=_SC_PARAMS,
)


def _mm_kern(x_ref, w_ref, o_ref):
    o_ref[...] = jnp.dot(x_ref[...], w_ref[...],
                         preferred_element_type=jnp.float32)


def _matmul(x, w0):
    m = x.shape[0]
    return pl.pallas_call(
        _mm_kern,
        grid=(m // RBLK,),
        in_specs=[
            pl.BlockSpec((RBLK, HID), lambda i: (i, 0)),
            pl.BlockSpec((HID, HID), lambda i: (0, 0)),
        ],
        out_specs=pl.BlockSpec((RBLK, HID), lambda i: (i, 0)),
        out_shape=jax.ShapeDtypeStruct((m, HID), jnp.float32),
    )(x, w0)


def _b_kern(y_ref, ns_ref, o_ref):
    y = y_ref[...]
    ns = ns_ref[...]
    for k in range(2):
        o_ref[k, 0] = y[:, k * 128:(k + 1) * 128] * ns


def _table1(y_cat, ns_cat):
    out = pl.pallas_call(
        _b_kern,
        grid=(2 * NRB,),
        in_specs=[
            pl.BlockSpec((RBLK, HID), lambda i: (i, 0)),
            pl.BlockSpec((RBLK, 1), lambda i: (i, 0)),
        ],
        out_specs=pl.BlockSpec((2, 1, RBLK, 128),
                               lambda i: (0, i // NRB, i % NRB, 0)),
        out_shape=jax.ShapeDtypeStruct((2, 2, NP, 128), jnp.float32),
    )(y_cat, ns_cat)
    return out.reshape(2, 2 * NP, 128)


def _d_kern(a_ref, nd_ref, ns_ref, b0_ref, o_ref):
    nd = nd_ref[...]
    ns = ns_ref[...]
    for k in range(2):
        b0row = b0_ref[k, :]
        for c in range(2):
            h = jnp.maximum(a_ref[k, c] * nd + b0row[None, :], 0.0)
            o_ref[k, c] = h * ns


def _table2(agg1, nd, ns, b0r):
    a = agg1.reshape(2, 2, NP, 128)
    out = pl.pallas_call(
        _d_kern,
        grid=(NRB,),
        in_specs=[
            pl.BlockSpec((2, 2, RBLK, 128), lambda i: (0, 0, i, 0)),
            pl.BlockSpec((RBLK, 1), lambda i: (i, 0)),
            pl.BlockSpec((RBLK, 1), lambda i: (i, 0)),
            pl.BlockSpec((2, 128), lambda i: (0, 0)),
        ],
        out_specs=pl.BlockSpec((2, 2, RBLK, 128), lambda i: (0, 0, i, 0)),
        out_shape=jax.ShapeDtypeStruct((2, 2, NP, 128), jnp.float32),
    )(a, nd, ns, b0r)
    return out.reshape(2, 2 * NP, 128)


def _f1_kern(a_ref, nd_ref, o_ref):
    @pl.when(pl.program_id(0) == 0)
    def _():
        o_ref[...] = jnp.zeros_like(o_ref)

    nd = nd_ref[...]
    for k in range(2):
        o_ref[k, :] += jnp.sum(a_ref[k, 0] * nd, axis=0)


def _colmean(agg2, nd):
    a = agg2.reshape(2, 2, NP, 128)
    return pl.pallas_call(
        _f1_kern,
        grid=(NRB,),
        in_specs=[
            pl.BlockSpec((2, 1, RBLK, 128), lambda i: (0, 0, i, 0)),
            pl.BlockSpec((RBLK, 1), lambda i: (i, 0)),
        ],
        out_specs=pl.BlockSpec((2, 128), lambda i: (0, 0)),
        out_shape=jax.ShapeDtypeStruct((2, 128), jnp.float32),
    )(a, nd)


def _f2_kern(m_ref, w1_ref, b1_ref, dw_ref, v_ref, c_ref):
    m = jnp.concatenate([m_ref[0:1, :], m_ref[1:2, :]], axis=1) * (1.0 / N)
    s_lin = jnp.dot(m, w1_ref[...], preferred_element_type=jnp.float32) \
        + b1_ref[...]
    summ = 1.0 / (1.0 + jnp.exp(-s_lin))
    dims = (((1,), (1,)), ((), ()))
    ws = lax.dot_general(summ, dw_ref[...], dims,
                         preferred_element_type=jnp.float32)
    v = lax.dot_general(ws, w1_ref[...], dims,
                        preferred_element_type=jnp.float32)
    cc = lax.dot_general(ws, b1_ref[...], dims,
                         preferred_element_type=jnp.float32)
    v_ref[...] = v
    c_ref[...] = cc


def _disc(m2, w1, b1r, dw):
    return pl.pallas_call(
        _f2_kern,
        in_specs=[
            pl.BlockSpec((2, 128), lambda: (0, 0)),
            pl.BlockSpec((HID, HID), lambda: (0, 0)),
            pl.BlockSpec((1, HID), lambda: (0, 0)),
            pl.BlockSpec((HID, HID), lambda: (0, 0)),
        ],
        out_specs=[
            pl.BlockSpec((1, HID), lambda: (0, 0)),
            pl.BlockSpec((1, 1), lambda: (0, 0)),
        ],
        out_shape=[
            jax.ShapeDtypeStruct((1, HID), jnp.float32),
            jax.ShapeDtypeStruct((1, 1), jnp.float32),
        ],
    )(m2, w1, b1r, dw)


def _f3_kern(a_ref, nd_ref, v_ref, c_ref, o_ref):
    @pl.when(pl.program_id(0) == 0)
    def _():
        o_ref[...] = jnp.zeros_like(o_ref)

    nd = nd_ref[...]
    v = v_ref[...]
    dims = (((1,), (1,)), ((), ()))
    for c in range(2):
        score = c_ref[...]
        for k in range(2):
            score += lax.dot_general(a_ref[k, c] * nd,
                                     v[:, k * 128:(k + 1) * 128], dims,
                                     preferred_element_type=jnp.float32)
        x = -score if c == 0 else score
        sp = jnp.maximum(x, 0.0) + jnp.log(1.0 + jnp.exp(-jnp.abs(x)))
        o_ref[...] += jnp.sum(sp, axis=(0, 1), keepdims=True)


def _loss(agg2, nd, v, cc):
    a = agg2.reshape(2, 2, NP, 128)
    return pl.pallas_call(
        _f3_kern,
        grid=(NRB,),
        in_specs=[
            pl.BlockSpec((2, 2, RBLK, 128), lambda i: (0, 0, i, 0)),
            pl.BlockSpec((RBLK, 1), lambda i: (i, 0)),
            pl.BlockSpec((1, HID), lambda i: (0, 0)),
            pl.BlockSpec((1, 1), lambda i: (0, 0)),
        ],
        out_specs=pl.BlockSpec((1, 1), lambda i: (0, 0)),
        out_shape=jax.ShapeDtypeStruct((1, 1), jnp.float32),
    )(a, nd, v, cc)


def kernel(features, edge_index, W0, b0, W1, b1, disc_W):
    src = edge_index[0]
    dst = edge_index[1]
    perm_arr, _ = _perm_arrays()
    perm_pad = jnp.concatenate(
        [perm_arr, jnp.zeros((NP - N,), jnp.int32)]).reshape(NC, NT, 4, BATCH)

    edges3 = edge_index.reshape(2, NT, NB, BATCH)
    z128 = jnp.zeros((NP, 128), jnp.float32)
    ones128 = jnp.ones((BATCH, 128), jnp.float32)

    y0 = _matmul(features, W0)
    deg2, xp = _prep(features, perm_pad, edges3, z128, ones128)

    def norm(deg):
        return jnp.where(deg > 0.0, lax.rsqrt(jnp.maximum(deg, 1.0)), 0.0)

    ns = norm(deg2[0, :N, 0])
    nd = norm(deg2[1, :N, 0])
    ns2 = ns[:, None]
    nd2 = nd[:, None]

    dst3 = dst.reshape(NT, NB, BATCH)
    srcs = jnp.concatenate([src, src + NP])

    y1 = _matmul(xp[:N], W0)
    y_cat = jnp.concatenate([y0, y1], axis=0)
    ns_cat = jnp.concatenate([ns2, ns2], axis=0)

    t1 = _table1(y_cat, ns_cat)
    agg1 = _edge_pass(t1, srcs, dst3, z128)
    t2 = _table2(agg1, nd2, ns2, b0.reshape(2, 128))
    agg2 = _edge_pass(t2, srcs, dst3, z128)

    m2 = _colmean(agg2, nd2)
    v, cc = _disc(m2, W1, b1.reshape(1, HID), disc_W)
    total = _loss(agg2, nd2, v, cc)
    return total[0, 0] * (1.0 / N)


# speedup vs baseline: 1.0287x; 1.0287x over previous
import jax
import jax.numpy as jnp
import numpy as np
from jax import lax
from jax.experimental import pallas as pl
from jax.experimental.pallas import tpu as pltpu
from jax.experimental.pallas import tpu_sc as plsc

N = 10000
NP = 10240
E = 160000
HID = 256
NT = 16
NC = 2
ROWS_T = NP // NT
EP_T = E // NT
BATCH = 80
NB = EP_T // BATCH
RBLK = 1000
NRB = N // RBLK

try:
    _PERM = np.asarray(
        jax.device_get(jax.random.permutation(jax.random.key(42), N)),
        dtype=np.int32)
    _PERM_INV = np.argsort(_PERM).astype(np.int32)
except Exception:
    _PERM = None
    _PERM_INV = None


def _perm_arrays():
    if _PERM is not None:
        return jnp.asarray(_PERM), None
    perm = jax.random.permutation(jax.random.key(42), N).astype(jnp.int32)
    return perm, None


_MESH = plsc.VectorSubcoreMesh(core_axis_name="c", subcore_axis_name="s")
_SC_PARAMS = pltpu.CompilerParams(needs_layout_passes=False)


def _prep_body(feat_hbm, permb_hbm, edges3_hbm, z128_hbm, ones_hbm,
               deg2_hbm, xp_hbm,
               pidx_v, prow_v, idx_v, ones_v, acc, psem):
    c = lax.axis_index("c")
    s = lax.axis_index("s")
    wid = c * NT + s

    pltpu.sync_copy(permb_hbm.at[c, s], pidx_v)
    for b in range(4):
        pltpu.async_copy(feat_hbm.at[pidx_v.at[b]], prow_v, psem).wait()
        pltpu.sync_copy(prow_v,
                        xp_hbm.at[pl.ds(wid * 320 + b * BATCH, BATCH)])

    pltpu.sync_copy(ones_hbm, ones_v)
    pltpu.sync_copy(z128_hbm.at[pl.ds(s * ROWS_T, ROWS_T)],
                    acc.at[pl.ds(s * ROWS_T, ROWS_T)])
    pltpu.sync_copy(edges3_hbm.at[c, s], idx_v)
    plsc.subcore_barrier()

    def dbody(b, carry):
        pltpu.sync_copy(ones_v, acc.at[idx_v.at[b]], add=True)
        return carry

    lax.fori_loop(0, NB, dbody, 0)
    plsc.subcore_barrier()
    pltpu.sync_copy(acc.at[pl.ds(s * ROWS_T, ROWS_T)],
                    deg2_hbm.at[c, pl.ds(s * ROWS_T, ROWS_T)])


_prep = pl.kernel(
    _prep_body,
    out_type=[
        jax.ShapeDtypeStruct((NC, NP, 128), jnp.float32),
        jax.ShapeDtypeStruct((NP, HID), jnp.float32),
    ],
    mesh=_MESH,
    scratch_types=[
        pltpu.VMEM((4, BATCH), jnp.int32),
        pltpu.VMEM((BATCH, HID), jnp.float32),
        pltpu.VMEM((NB, BATCH), jnp.int32),
        pltpu.VMEM((BATCH, 128), jnp.float32),
        pltpu.VMEM_SHARED((NP, 128), jnp.float32),
        pltpu.SemaphoreType.DMA,
    ],
    compiler_params=_SC_PARAMS,
)


def _edge_body(t_hbm, srcs_hbm, dst3_hbm, z128_hbm,
               agg_hbm,
               src_v, dst_v, rows0, rows1, acc, sem0, sem1):
    c = lax.axis_index("c")
    s = lax.axis_index("s")
    pltpu.sync_copy(srcs_hbm.at[pl.ds(c * E + s * EP_T, EP_T)], src_v)
    pltpu.sync_copy(dst3_hbm.at[s], dst_v)
    rows = (rows0, rows1)
    sems = (sem0, sem1)

    for k in range(2):
        tbl = t_hbm.at[k]
        pltpu.sync_copy(z128_hbm.at[pl.ds(s * ROWS_T, ROWS_T)],
                        acc.at[pl.ds(s * ROWS_T, ROWS_T)])
        plsc.subcore_barrier()

        pltpu.async_copy(tbl.at[src_v.at[pl.ds(0, BATCH)]], rows0, sem0)

        def pair(i, carry):
            for j in range(2):
                b = i * 2 + j

                @pl.when(b + 1 < NB)
                def _():
                    pltpu.async_copy(
                        tbl.at[src_v.at[pl.ds((b + 1) * BATCH, BATCH)]],
                        rows[1 - j], sems[1 - j])

                pltpu.make_async_copy(
                    tbl.at[src_v.at[pl.ds(b * BATCH, BATCH)]],
                    rows[j], sems[j]).wait()
                pltpu.sync_copy(rows[j], acc.at[dst_v.at[b]], add=True)
            return carry

        lax.fori_loop(0, NB // 2, pair, 0)
        b_last = NB - 1
        pltpu.make_async_copy(
            tbl.at[src_v.at[pl.ds(b_last * BATCH, BATCH)]],
            rows0, sem0).wait()
        pltpu.sync_copy(rows0, acc.at[dst_v.at[b_last]], add=True)

        plsc.subcore_barrier()
        row0 = c * NP + s * ROWS_T
        pltpu.sync_copy(acc.at[pl.ds(s * ROWS_T, ROWS_T)],
                        agg_hbm.at[k, pl.ds(row0, ROWS_T)])
        plsc.subcore_barrier()


_edge_pass = pl.kernel(
    _edge_body,
    out_type=jax.ShapeDtypeStruct((2, 2 * NP, 128), jnp.float32),
    mesh=_MESH,
    scratch_types=[
        pltpu.VMEM((EP_T,), jnp.int32),
        pltpu.VMEM((NB, BATCH), jnp.int32),
        pltpu.VMEM((BATCH, 128), jnp.float32),
        pltpu.VMEM((BATCH, 128), jnp.float32),
        pltpu.VMEM_SHARED((NP, 128), jnp.float32),
        pltpu.SemaphoreType.DMA,
        pltpu.SemaphoreType.DMA,
    ],
    compiler_params=_SC_PARAMS,
)


def _b_kern(x_ref, w_ref, ns_ref, o_ref):
    y = jnp.dot(x_ref[...], w_ref[...], preferred_element_type=jnp.float32)
    ns = ns_ref[...]
    for k in range(2):
        o_ref[k, 0] = y[:, k * 128:(k + 1) * 128] * ns


def _table1(x_cat, w0, ns_cat):
    out = pl.pallas_call(
        _b_kern,
        grid=(2 * NRB,),
        in_specs=[
            pl.BlockSpec((RBLK, HID), lambda i: (i, 0)),
            pl.BlockSpec((HID, HID), lambda i: (0, 0)),
            pl.BlockSpec((RBLK, 1), lambda i: (i, 0)),
        ],
        out_specs=pl.BlockSpec((2, 1, RBLK, 128),
                               lambda i: (0, i // NRB, i % NRB, 0)),
        out_shape=jax.ShapeDtypeStruct((2, 2, NP, 128), jnp.float32),
    )(x_cat, w0, ns_cat)
    return out.reshape(2, 2 * NP, 128)


def _d_kern(a_ref, nd_ref, ns_ref, b0_ref, o_ref):
    nd = nd_ref[...]
    ns = ns_ref[...]
    for k in range(2):
        b0row = b0_ref[k, :]
        for c in range(2):
            h = jnp.maximum(a_ref[k, c] * nd + b0row[None, :], 0.0)
            o_ref[k, c] = h * ns


def _table2(agg1, nd, ns, b0r):
    a = agg1.reshape(2, 2, NP, 128)
    out = pl.pallas_call(
        _d_kern,
        grid=(NRB,),
        in_specs=[
            pl.BlockSpec((2, 2, RBLK, 128), lambda i: (0, 0, i, 0)),
            pl.BlockSpec((RBLK, 1), lambda i: (i, 0)),
            pl.BlockSpec((RBLK, 1), lambda i: (i, 0)),
            pl.BlockSpec((2, 128), lambda i: (0, 0)),
        ],
        out_specs=pl.BlockSpec((2, 2, RBLK, 128), lambda i: (0, 0, i, 0)),
        out_shape=jax.ShapeDtypeStruct((2, 2, NP, 128), jnp.float32),
    )(a, nd, ns, b0r)
    return out.reshape(2, 2 * NP, 128)


def _f1_kern(a_ref, nd_ref, w1_ref, b1_ref, dw_ref, v_ref, c_ref, m_acc):
    i = pl.program_id(0)

    @pl.when(i == 0)
    def _():
        m_acc[...] = jnp.zeros_like(m_acc)

    nd = nd_ref[...]
    for k in range(2):
        m_acc[k, :] += jnp.sum(a_ref[k, 0] * nd, axis=0)

    @pl.when(i == NRB - 1)
    def _():
        m = jnp.concatenate([m_acc[0:1, :], m_acc[1:2, :]],
                            axis=1) * (1.0 / N)
        s_lin = jnp.dot(m, w1_ref[...],
                        preferred_element_type=jnp.float32) + b1_ref[...]
        summ = 1.0 / (1.0 + jnp.exp(-s_lin))
        dims = (((1,), (1,)), ((), ()))
        ws = lax.dot_general(summ, dw_ref[...], dims,
                             preferred_element_type=jnp.float32)
        v_ref[...] = lax.dot_general(ws, w1_ref[...], dims,
                                     preferred_element_type=jnp.float32)
        c_ref[...] = lax.dot_general(ws, b1_ref[...], dims,
                                     preferred_element_type=jnp.float32)


def _colmean_disc(agg2, nd, w1, b1r, dw):
    a = agg2.reshape(2, 2, NP, 128)
    return pl.pallas_call(
        _f1_kern,
        grid=(NRB,),
        in_specs=[
            pl.BlockSpec((2, 1, RBLK, 128), lambda i: (0, 0, i, 0)),
            pl.BlockSpec((RBLK, 1), lambda i: (i, 0)),
            pl.BlockSpec((HID, HID), lambda i: (0, 0)),
            pl.BlockSpec((1, HID), lambda i: (0, 0)),
            pl.BlockSpec((HID, HID), lambda i: (0, 0)),
        ],
        out_specs=[
            pl.BlockSpec((1, HID), lambda i: (0, 0)),
            pl.BlockSpec((1, 1), lambda i: (0, 0)),
        ],
        out_shape=[
            jax.ShapeDtypeStruct((1, HID), jnp.float32),
            jax.ShapeDtypeStruct((1, 1), jnp.float32),
        ],
        scratch_shapes=[pltpu.VMEM((2, 128), jnp.float32)],
    )(a, nd, w1, b1r, dw)


def _f3_kern(a_ref, nd_ref, v_ref, c_ref, o_ref):
    @pl.when(pl.program_id(0) == 0)
    def _():
        o_ref[...] = jnp.zeros_like(o_ref)

    nd = nd_ref[...]
    v = v_ref[...]
    dims = (((1,), (1,)), ((), ()))
    for c in range(2):
        score = c_ref[...]
        for k in range(2):
            score += lax.dot_general(a_ref[k, c] * nd,
                                     v[:, k * 128:(k + 1) * 128], dims,
                                     preferred_element_type=jnp.float32)
        x = -score if c == 0 else score
        sp = jnp.maximum(x, 0.0) + jnp.log(1.0 + jnp.exp(-jnp.abs(x)))
        o_ref[...] += jnp.sum(sp, axis=(0, 1), keepdims=True)


def _loss(agg2, nd, v, cc):
    a = agg2.reshape(2, 2, NP, 128)
    return pl.pallas_call(
        _f3_kern,
        grid=(NRB,),
        in_specs=[
            pl.BlockSpec((2, 2, RBLK, 128), lambda i: (0, 0, i, 0)),
            pl.BlockSpec((RBLK, 1), lambda i: (i, 0)),
            pl.BlockSpec((1, HID), lambda i: (0, 0)),
            pl.BlockSpec((1, 1), lambda i: (0, 0)),
        ],
        out_specs=pl.BlockSpec((1, 1), lambda i: (0, 0)),
        out_shape=jax.ShapeDtypeStruct((1, 1), jnp.float32),
    )(a, nd, v, cc)


def kernel(features, edge_index, W0, b0, W1, b1, disc_W):
    src = edge_index[0]
    dst = edge_index[1]
    perm_arr, _ = _perm_arrays()
    perm_pad = jnp.concatenate(
        [perm_arr, jnp.zeros((NP - N,), jnp.int32)]).reshape(NC, NT, 4, BATCH)

    edges3 = edge_index.reshape(2, NT, NB, BATCH)
    z128 = jnp.zeros((NP, 128), jnp.float32)
    ones128 = jnp.ones((BATCH, 128), jnp.float32)

    deg2, xp = _prep(features, perm_pad, edges3, z128, ones128)

    def norm(deg):
        return jnp.where(deg > 0.0, lax.rsqrt(jnp.maximum(deg, 1.0)), 0.0)

    ns = norm(deg2[0, :N, 0])
    nd = norm(deg2[1, :N, 0])
    ns2 = ns[:, None]
    nd2 = nd[:, None]

    dst3 = dst.reshape(NT, NB, BATCH)
    srcs = jnp.concatenate([src, src + NP])

    x_cat = jnp.concatenate([features, xp[:N]], axis=0)
    ns_cat = jnp.concatenate([ns2, ns2], axis=0)

    t1 = _table1(x_cat, W0, ns_cat)
    agg1 = _edge_pass(t1, srcs, dst3, z128)
    t2 = _table2(agg1, nd2, ns2, b0.reshape(2, 128))
    agg2 = _edge_pass(t2, srcs, dst3, z128)

    v, cc = _colmean_disc(agg2, nd2, W1, b1.reshape(1, HID), disc_W)
    total = _loss(agg2, nd2, v, cc)
    return total[0, 0] * (1.0 / N)
